# Initial kernel scaffold; baseline (speedup 1.0000x reference)
import jax
import jax.numpy as jnp
from jax.experimental import pallas as pl


def kernel(x, edge_index, W1, b1, W2, b2):
    raise NotImplementedError("write your pallas kernel here")


# speedup vs baseline: 10.6637x; 10.6637x over previous
import functools

import jax
import jax.numpy as jnp
from jax import lax
from jax.experimental import pallas as pl
from jax.experimental.pallas import tpu as pltpu
from jax.experimental.pallas import tpu_sc as plsc

NC = 2
NS = 16
NW = NC * NS
CHUNK = 128
LANES = 16
DEGW = 128


def _fill(ref, rows, cols, value):
    @pl.loop(0, rows)
    def _(r):
        for c0 in range(0, cols, LANES):
            ref[r, pl.ds(c0, LANES)] = jnp.full((LANES,), value, jnp.float32)


def _make_deg_kernel(n_pad, e_pad):
    cpw = e_pad // NW // CHUNK
    epw = e_pad // NW
    rps = n_pad // NS
    mesh = plsc.VectorSubcoreMesh(core_axis_name="c", subcore_axis_name="s")

    @functools.partial(
        pl.kernel, mesh=mesh,
        out_type=jax.ShapeDtypeStruct((NC, n_pad, DEGW), jnp.float32),
        scratch_types=[
            pltpu.VMEM((CHUNK,), jnp.int32),
            pltpu.VMEM((CHUNK, DEGW), jnp.float32),
            pltpu.VMEM((CHUNK, DEGW), jnp.float32),
            pltpu.VMEM_SHARED((n_pad, DEGW), jnp.float32),
        ],
    )
    def deg_kernel(dst_hbm, out_hbm, idx_v, ones_v, zbuf, acc):
        c = lax.axis_index("c")
        s = lax.axis_index("s")
        wid = s * NC + c
        _fill(ones_v, CHUNK, DEGW, 1.0)
        _fill(zbuf, CHUNK, DEGW, 0.0)
        for k in range(rps // CHUNK):
            pltpu.sync_copy(zbuf, acc.at[pl.ds(s * rps + k * CHUNK, CHUNK)])
        plsc.subcore_barrier()
        base = wid * epw

        @pl.loop(0, cpw)
        def _(j):
            pltpu.sync_copy(dst_hbm.at[pl.ds(base + j * CHUNK, CHUNK)], idx_v)
            pltpu.sync_copy(ones_v, acc.at[idx_v], add=True)

        plsc.subcore_barrier()
        pltpu.sync_copy(acc.at[pl.ds(s * rps, rps)],
                        out_hbm.at[c].at[pl.ds(s * rps, rps)])

    return deg_kernel


def _make_seg_sum_kernel(n_pad, e_pad, d):
    cpw = e_pad // NW // CHUNK
    epw = e_pad // NW
    rps = n_pad // NS
    mesh = plsc.VectorSubcoreMesh(core_axis_name="c", subcore_axis_name="s")

    @functools.partial(
        pl.kernel, mesh=mesh,
        out_type=jax.ShapeDtypeStruct((NC, n_pad, d), jnp.float32),
        scratch_types=[
            pltpu.VMEM((CHUNK,), jnp.int32),
            pltpu.VMEM((CHUNK,), jnp.int32),
            pltpu.VMEM((CHUNK, d), jnp.float32),
            pltpu.VMEM((CHUNK, d), jnp.float32),
            pltpu.VMEM_SHARED((n_pad, d), jnp.float32),
            pltpu.SemaphoreType.DMA,
        ],
    )
    def seg_kernel(h_hbm, src_hbm, dst_hbm, out_hbm,
                   sidx, didx, rows, zbuf, acc, sem):
        c = lax.axis_index("c")
        s = lax.axis_index("s")
        wid = s * NC + c
        _fill(zbuf, CHUNK, d, 0.0)
        for k in range(rps // CHUNK):
            pltpu.sync_copy(zbuf, acc.at[pl.ds(s * rps + k * CHUNK, CHUNK)])
        plsc.subcore_barrier()
        base = wid * epw

        @pl.loop(0, cpw)
        def _(j):
            off = base + j * CHUNK
            pltpu.sync_copy(src_hbm.at[pl.ds(off, CHUNK)], sidx)
            pltpu.async_copy(h_hbm.at[sidx], rows, sem).wait()
            pltpu.sync_copy(dst_hbm.at[pl.ds(off, CHUNK)], didx)
            pltpu.sync_copy(rows, acc.at[didx], add=True)

        plsc.subcore_barrier()
        pltpu.sync_copy(acc.at[pl.ds(s * rps, rps)],
                        out_hbm.at[c].at[pl.ds(s * rps, rps)])

    return seg_kernel


def _dinv_block(deg_blk):
    deg = deg_blk[0, :, 0] + deg_blk[1, :, 0] + 1.0
    return lax.rsqrt(deg)


def _tc_scale_matmul(x, w, degp, bn=1000):
    n, f = x.shape
    hid = w.shape[1]

    def body(x_ref, w_ref, deg_ref, o_ref):
        dinv = _dinv_block(deg_ref[...])
        h = jnp.dot(x_ref[...], w_ref[...], preferred_element_type=jnp.float32)
        o_ref[...] = h * dinv[:, None]

    return pl.pallas_call(
        body,
        grid=(n // bn,),
        in_specs=[
            pl.BlockSpec((bn, f), lambda i: (i, 0)),
            pl.BlockSpec((f, hid), lambda i: (0, 0)),
            pl.BlockSpec((NC, bn, DEGW), lambda i: (0, i, 0)),
        ],
        out_specs=pl.BlockSpec((bn, hid), lambda i: (i, 0)),
        out_shape=jax.ShapeDtypeStruct((n, hid), jnp.float32),
    )(x, w, degp)


def _tc_mid(accp, h1p, degp, b1, w2p, bn=1000):
    n, hid = h1p.shape
    c_pad = w2p.shape[1]

    def body(a_ref, h_ref, deg_ref, b_ref, w_ref, o_ref):
        dinv = _dinv_block(deg_ref[...])
        a = a_ref[...]
        z = dinv[:, None] * (a[0] + a[1] + h_ref[...]) + b_ref[...]
        z = jnp.maximum(z, 0.0)
        o_ref[...] = jnp.dot(z, w_ref[...],
                             preferred_element_type=jnp.float32) * dinv[:, None]

    n_pad = accp.shape[1]
    return pl.pallas_call(
        body,
        grid=(n // bn,),
        in_specs=[
            pl.BlockSpec((NC, bn, hid), lambda i: (0, i, 0)),
            pl.BlockSpec((bn, hid), lambda i: (i, 0)),
            pl.BlockSpec((NC, bn, DEGW), lambda i: (0, i, 0)),
            pl.BlockSpec((1, hid), lambda i: (0, 0)),
            pl.BlockSpec((hid, c_pad), lambda i: (0, 0)),
        ],
        out_specs=pl.BlockSpec((bn, c_pad), lambda i: (i, 0)),
        out_shape=jax.ShapeDtypeStruct((n, c_pad), jnp.float32),
    )(accp, h1p, degp, b1, w2p)


def _tc_final(accp, h2p, degp, b2p, c_real, bn=1000):
    n, c_pad = h2p.shape

    def body(a_ref, h_ref, deg_ref, b_ref, o_ref):
        dinv = _dinv_block(deg_ref[...])
        a = a_ref[...]
        z = dinv[:, None] * (a[0] + a[1] + h_ref[...]) + b_ref[...]
        col = lax.broadcasted_iota(jnp.int32, (bn, c_pad), 1)
        mask = col < c_real
        zm = jnp.where(mask, z, -1e30)
        m = jnp.max(zm, axis=1, keepdims=True)
        e = jnp.where(mask, jnp.exp(zm - m), 0.0)
        lse = jnp.log(jnp.sum(e, axis=1, keepdims=True)) + m
        o_ref[...] = z - lse

    return pl.pallas_call(
        body,
        grid=(n // bn,),
        in_specs=[
            pl.BlockSpec((NC, bn, c_pad), lambda i: (0, i, 0)),
            pl.BlockSpec((bn, c_pad), lambda i: (i, 0)),
            pl.BlockSpec((NC, bn, DEGW), lambda i: (0, i, 0)),
            pl.BlockSpec((1, c_pad), lambda i: (0, 0)),
        ],
        out_specs=pl.BlockSpec((bn, c_pad), lambda i: (i, 0)),
        out_shape=jax.ShapeDtypeStruct((n, c_pad), jnp.float32),
    )(accp, h2p, degp, b2p)


def kernel(x, edge_index, W1, b1, W2, b2):
    n, f_in = x.shape
    hid = W1.shape[1]
    c_real = W2.shape[1]
    e = edge_index.shape[1]

    e_pad = -(-e // (NW * CHUNK)) * (NW * CHUNK)
    n_pad = -(-n // (NS * CHUNK)) * (NS * CHUNK)
    c_pad = 128

    src = edge_index[0]
    dst = edge_index[1]
    pad = e_pad - e
    src_p = jnp.concatenate([src, jnp.zeros((pad,), jnp.int32)])
    dst_p = jnp.concatenate([dst, jnp.full((pad,), n, jnp.int32)])

    w2p = jnp.zeros((hid, c_pad), jnp.float32).at[:, :c_real].set(W2)
    b1r = b1.reshape(1, hid)
    b2p = jnp.zeros((1, c_pad), jnp.float32).at[0, :c_real].set(b2)

    degp = _make_deg_kernel(n_pad, e_pad)(dst_p)

    h1p = _tc_scale_matmul(x, W1, degp)
    acc1 = _make_seg_sum_kernel(n_pad, e_pad, hid)(h1p, src_p, dst_p)
    h2p = _tc_mid(acc1, h1p, degp, b1r, w2p)
    acc2 = _make_seg_sum_kernel(n_pad, e_pad, c_pad)(h2p, src_p, dst_p)
    out = _tc_final(acc2, h2p, degp, b2p, c_real)
    return out[:, :c_real]
